# baseline (device time: 10471 ns/iter reference)
import jax
import jax.numpy as jnp
from jax import lax
from jax.experimental import pallas as pl
from jax.experimental.pallas import tpu as pltpu

N_DEV = 4
N_EXP = 8
E_LOCAL = 2
N_TOK = 256
D_IN = 128
D_OUT = 256
ROWS = N_TOK // N_DEV


def kernel(x, router_W, route_idx, expert_W, shared_W):
    def body(x_ref, rw_ref, idx_ref, ew_ref, sw_ref, out_ref,
             partial_ref, recv_ref, send_sems, recv_sems):
        my = lax.axis_index("i")

        bsem = pltpu.get_barrier_semaphore()
        for d in range(1, N_DEV):
            pl.semaphore_signal(
                bsem, inc=1,
                device_id=(lax.rem(my + d, N_DEV),),
                device_id_type=pl.DeviceIdType.MESH,
            )
        pl.semaphore_wait(bsem, N_DEV - 1)

        xv = x_ref[:, :]
        scores = jnp.dot(xv, rw_ref[:, :], preferred_element_type=jnp.float32)
        m = jnp.max(scores, axis=1, keepdims=True)
        p = jnp.exp(scores - m)
        probs = p / jnp.sum(p, axis=1, keepdims=True)
        eids = lax.broadcasted_iota(jnp.int32, (N_TOK, N_EXP), 1)
        gated = jnp.where(eids == idx_ref[:, :], probs, 0.0)

        partial = jnp.zeros((N_TOK, D_OUT), jnp.float32)
        for l in range(E_LOCAL):
            g = my * E_LOCAL + l
            gate = jnp.sum(jnp.where(eids == g, gated, 0.0), axis=1,
                           keepdims=True)
            partial += jnp.dot(xv * gate, ew_ref[l, :, :],
                               preferred_element_type=jnp.float32)
        partial_ref[...] = partial.reshape(N_DEV, ROWS, D_OUT)

        rdmas = []
        for d in range(1, N_DEV):
            tgt = lax.rem(my + d, N_DEV)
            rdma = pltpu.make_async_remote_copy(
                src_ref=partial_ref.at[tgt],
                dst_ref=recv_ref.at[d - 1],
                send_sem=send_sems.at[d - 1],
                recv_sem=recv_sems.at[d - 1],
                device_id=(tgt,),
                device_id_type=pl.DeviceIdType.MESH,
            )
            rdma.start()
            rdmas.append(rdma)

        x_my = x_ref[pl.ds(my * ROWS, ROWS), :]
        acc = jnp.dot(x_my, sw_ref[:, :], preferred_element_type=jnp.float32)
        acc = acc + partial_ref[my]

        for r in rdmas:
            r.wait_recv()
        acc = acc + recv_ref[0] + recv_ref[1] + recv_ref[2]
        out_ref[...] = acc

        for r in rdmas:
            r.wait_send()

    return pl.pallas_call(
        body,
        out_shape=jax.ShapeDtypeStruct((ROWS, D_OUT), jnp.float32),
        in_specs=[pl.BlockSpec(memory_space=pltpu.VMEM)] * 5,
        out_specs=pl.BlockSpec(memory_space=pltpu.VMEM),
        scratch_shapes=[
            pltpu.VMEM((N_DEV, ROWS, D_OUT), jnp.float32),
            pltpu.VMEM((N_DEV - 1, ROWS, D_OUT), jnp.float32),
            pltpu.SemaphoreType.DMA((N_DEV - 1,)),
            pltpu.SemaphoreType.DMA((N_DEV - 1,)),
        ],
        compiler_params=pltpu.CompilerParams(collective_id=0),
    )(x, router_W, route_idx, expert_W, shared_W)


# device time: 7813 ns/iter; 1.3402x vs baseline; 1.3402x over previous
import jax
import jax.numpy as jnp
from jax import lax
from jax.experimental import pallas as pl
from jax.experimental.pallas import tpu as pltpu

N_DEV = 4
N_EXP = 8
E_LOCAL = 2
N_TOK = 256
D_IN = 128
D_OUT = 256
ROWS = N_TOK // N_DEV


def kernel(x, router_W, route_idx, expert_W, shared_W):
    def body(x_ref, rwt_hbm, idx_hbm, ew_hbm, sw_hbm, out_ref,
             rwt_ref, idx_ref, ew_ref, sw_ref,
             gate_ref, partial_ref, recv_ref,
             load_sems, send_sems, recv_sems):
        my = lax.axis_index("i")

        bsem = pltpu.get_barrier_semaphore()
        for d in range(1, N_DEV):
            pl.semaphore_signal(
                bsem, inc=1,
                device_id=(lax.rem(my + d, N_DEV),),
                device_id_type=pl.DeviceIdType.MESH,
            )

        rw_load = pltpu.make_async_copy(rwt_hbm, rwt_ref, load_sems.at[2])
        rw_load.start()
        idx_load = pltpu.make_async_copy(idx_hbm, idx_ref, load_sems.at[3])
        idx_load.start()
        ew_load = pltpu.make_async_copy(ew_hbm, ew_ref, load_sems.at[0])
        ew_load.start()
        sw_load = pltpu.make_async_copy(sw_hbm, sw_ref, load_sems.at[1])
        sw_load.start()
        rw_load.wait()
        idx_load.wait()

        xv = x_ref[:, :]
        scores = lax.dot_general(
            xv, rwt_ref[:, :], (((1,), (1,)), ((), ())),
            preferred_element_type=jnp.float32)
        m = jnp.max(scores, axis=1, keepdims=True)
        p = jnp.exp(scores - m)
        probs = p / jnp.sum(p, axis=1, keepdims=True)
        eids = lax.broadcasted_iota(jnp.int32, (N_TOK, N_EXP), 1)
        idx_col = jnp.reshape(idx_ref[:, :], (N_TOK, 1))
        gated = jnp.where(eids == idx_col, probs, 0.0)
        for l in range(E_LOCAL):
            g = my * E_LOCAL + l
            gate_ref[l] = jnp.sum(jnp.where(eids == g, gated, 0.0), axis=1,
                                  keepdims=True)

        ew_load.wait()

        rdmas = {}
        first = True
        for d in (2, 1, 3):
            tgt = lax.rem(my + d, N_DEV)
            rows = pl.ds(tgt * ROWS, ROWS)
            blk = (jnp.dot(x_ref[rows, :], ew_ref[0],
                           preferred_element_type=jnp.float32)
                   * gate_ref[0, rows, :]
                   + jnp.dot(x_ref[rows, :], ew_ref[1],
                             preferred_element_type=jnp.float32)
                   * gate_ref[1, rows, :])
            partial_ref[d - 1] = blk.astype(jnp.bfloat16)
            if first:
                pl.semaphore_wait(bsem, N_DEV - 1)
                first = False
            rdma = pltpu.make_async_remote_copy(
                src_ref=partial_ref.at[d - 1],
                dst_ref=recv_ref.at[d - 1],
                send_sem=send_sems.at[d - 1],
                recv_sem=recv_sems.at[d - 1],
                device_id=(tgt,),
                device_id_type=pl.DeviceIdType.MESH,
            )
            rdma.start()
            rdmas[d] = rdma

        sw_load.wait()
        rows = pl.ds(my * ROWS, ROWS)
        acc = (jnp.dot(x_ref[rows, :], sw_ref[:, :],
                       preferred_element_type=jnp.float32)
               + jnp.dot(x_ref[rows, :], ew_ref[0],
                         preferred_element_type=jnp.float32)
               * gate_ref[0, rows, :]
               + jnp.dot(x_ref[rows, :], ew_ref[1],
                         preferred_element_type=jnp.float32)
               * gate_ref[1, rows, :])

        for d in (1, 3, 2):
            rdmas[d].wait_recv()
            acc += recv_ref[d - 1].astype(jnp.float32)
        out_ref[...] = acc

        for d in (2, 1, 3):
            rdmas[d].wait_send()

    grid_spec = pl.pallas_call(
        body,
        out_shape=jax.ShapeDtypeStruct((ROWS, D_OUT), jnp.float32),
        in_specs=[
            pl.BlockSpec(memory_space=pltpu.VMEM),
            pl.BlockSpec(memory_space=pltpu.MemorySpace.HBM),
            pl.BlockSpec(memory_space=pltpu.MemorySpace.HBM),
            pl.BlockSpec(memory_space=pltpu.MemorySpace.HBM),
            pl.BlockSpec(memory_space=pltpu.MemorySpace.HBM),
        ],
        out_specs=pl.BlockSpec(memory_space=pltpu.VMEM),
        scratch_shapes=[
            pltpu.VMEM((N_EXP, D_IN), jnp.float32),
            pltpu.VMEM((1, N_TOK), jnp.int32),
            pltpu.VMEM((E_LOCAL, D_IN, D_OUT), jnp.float32),
            pltpu.VMEM((D_IN, D_OUT), jnp.float32),
            pltpu.VMEM((E_LOCAL, N_TOK, 1), jnp.float32),
            pltpu.VMEM((N_DEV - 1, ROWS, D_OUT), jnp.bfloat16),
            pltpu.VMEM((N_DEV - 1, ROWS, D_OUT), jnp.bfloat16),
            pltpu.SemaphoreType.DMA((4,)),
            pltpu.SemaphoreType.DMA((N_DEV - 1,)),
            pltpu.SemaphoreType.DMA((N_DEV - 1,)),
        ],
        compiler_params=pltpu.CompilerParams(collective_id=0),
    )
    return grid_spec(
        x,
        pltpu.with_memory_space_constraint(router_W.T, pltpu.MemorySpace.HBM),
        pltpu.with_memory_space_constraint(route_idx.reshape(1, N_TOK),
                                           pltpu.MemorySpace.HBM),
        pltpu.with_memory_space_constraint(expert_W, pltpu.MemorySpace.HBM),
        pltpu.with_memory_space_constraint(shared_W, pltpu.MemorySpace.HBM),
    )
